# baseline (speedup 1.0000x reference)
import math

import jax
import jax.numpy as jnp
import numpy as np
from jax.experimental import pallas as pl

N = 6000
NG = 8
H = 4


def _pos_encoding(n, d):
    pos = np.arange(n, dtype=np.float32)[:, None]
    div = np.exp(np.arange(0, d, 2, dtype=np.float32) * (-(math.log(10000.0) / d)))
    pe = np.zeros((n, d), dtype=np.float32)
    pe[:, 0::2] = np.sin(pos * div)
    pe[:, 1::2] = np.cos(pos * div)
    return jnp.asarray(pe)


def _seg_mean(x, seg, ng):
    s = jax.ops.segment_sum(x, seg, num_segments=ng)
    cnt = jax.ops.segment_sum(jnp.ones(x.shape[0], dtype=x.dtype), seg, num_segments=ng)
    return s / cnt[:, None]


def _gru(x, h, wih, whh, bih, bhh):
    gi = x @ wih.T + bih
    gh = h @ whh.T + bhh
    i_r, i_z, i_n = jnp.split(gi, 3, axis=-1)
    h_r, h_z, h_n = jnp.split(gh, 3, axis=-1)
    r = jax.nn.sigmoid(i_r + h_r)
    z = jax.nn.sigmoid(i_z + h_z)
    nn_ = jnp.tanh(i_n + r * h_n)
    return (1 - z) * nn_ + z * h


def kernel(x_wt, edge_index_wt, batch_wt, x_diff, edge_index_diff, batch_diff, params):
    p = params
    n = x_wt.shape[0]
    x = jax.nn.relu(x_wt @ p['ne_w'] + p['ne_b'])
    src, dst = edge_index_wt[0], edge_index_wt[1]
    for cname in ('conv0', 'conv1', 'conv2'):
        c = p[cname]
        xl = x @ c['lin_w'] + c['lin_b']
        dout = xl.shape[1]
        dh = dout // H
        Q = (xl @ c['q_w'] + c['q_b']).reshape(n, H, dh)
        K = (xl @ c['k_w'] + c['k_b']).reshape(n, H, dh)
        V = (xl @ c['v_w'] + c['v_b']).reshape(n, H, dh)
        q = Q[dst]
        k = K[src]
        v = V[src]
        att = jnp.einsum('ehd,egd->ehg', q, k) / (dh ** 0.5)
        att = jax.nn.softmax(att, axis=-1)
        out = jnp.einsum('ehg,egd->ehd', att, v).reshape(-1, dout)
        x = jax.nn.relu(jax.ops.segment_sum(out, dst, num_segments=n))
    sf = _seg_mean(x, batch_wt, NG)
    sf = jax.nn.relu(sf @ p['sft_w'] + p['sft_b'])
    h0 = jnp.zeros_like(sf)
    sf = _gru(sf, h0, p['gru_wih'], p['gru_whh'], p['gru_bih'], p['gru_bhh'])
    x = x + _pos_encoding(x.shape[0], x.shape[1])
    g_wt = jnp.concatenate([_seg_mean(x, batch_wt, NG), sf], axis=1)
    sd, dd = edge_index_diff[0], edge_index_diff[1]
    indeg = jax.ops.segment_sum(jnp.ones(dd.shape[0], jnp.float32), dd, num_segments=n)
    dinv = 1.0 / jnp.sqrt(indeg + 1.0)
    y = x_diff
    for i, (w, b) in enumerate([(p['gcn0_w'], p['gcn0_b']),
                                (p['gcn1_w'], p['gcn1_b']),
                                (p['gcn2_w'], p['gcn2_b'])]):
        xs = dinv[:, None] * (y @ w)
        acc = jax.ops.segment_sum(xs[sd], dd, num_segments=n)
        y = dinv[:, None] * (acc + xs) + b
        if i < 2:
            y = jax.nn.relu(y)
    g_diff = _seg_mean(y, batch_diff, NG)
    comb = jnp.concatenate([g_wt, g_diff], axis=-1)
    z = jax.nn.relu(comb @ p['mlp1_w'] + p['mlp1_b'])
    z = jax.nn.relu(z @ p['mlp2_w'] + p['mlp2_b'])
    z = z @ p['mlp3_w'] + p['mlp3_b']
    return z @ p['pred_w'] + p['pred_b']


# speedup vs baseline: 4.1032x; 4.1032x over previous
import functools
import math

import jax
import jax.numpy as jnp
import numpy as np
from jax import lax
from jax.experimental import pallas as pl
from jax.experimental.pallas import tpu as pltpu
from jax.experimental.pallas import tpu_sc as plsc

N = 6000
NP = 6016
NG = 8
H = 4
NC = 2
NS = 16
NW = NC * NS
C = 96
NH = 6144


def _pos_encoding(n, d):
    pos = np.arange(n, dtype=np.float32)[:, None]
    div = np.exp(np.arange(0, d, 2, dtype=np.float32) * (-(math.log(10000.0) / d)))
    pe = np.zeros((n, d), dtype=np.float32)
    pe[:, 0::2] = np.sin(pos * div)
    pe[:, 1::2] = np.cos(pos * div)
    return jnp.asarray(pe)


def _sc_mesh():
    return plsc.VectorSubcoreMesh(core_axis_name="c", subcore_axis_name="s")


def _bounce_chunks(rpt, cap=C):
    chunks = []
    left = rpt
    while left > 0:
        nb = min(cap, left)
        chunks.append(nb)
        left -= nb
    assert all(nb % 8 == 0 for nb in chunks)
    return chunks


def _sc_deg(dst, zeros_h):
    e = dst.shape[0]
    epw = e // NW
    nchunks = epw // C
    rpt = NH // NS

    @functools.partial(
        pl.kernel,
        out_type=jax.ShapeDtypeStruct((NC, NH), jnp.float32),
        mesh=_sc_mesh(),
        scratch_types=[
            pltpu.VMEM((C,), jnp.int32),
            pltpu.VMEM((C,), jnp.float32),
            pltpu.VMEM_SHARED((NH,), jnp.float32),
        ],
    )
    def k(dst_h, z_h, out_h, didx, ones_v, hist):
        cid = lax.axis_index("c")
        sid = lax.axis_index("s")
        for i in range(C // 16):
            ones_v[pl.ds(i * 16, 16)] = jnp.ones((16,), jnp.float32)
        pltpu.sync_copy(z_h.at[pl.ds(sid * rpt, rpt)], hist.at[pl.ds(sid * rpt, rpt)])
        plsc.subcore_barrier()
        base = (cid * NS + sid) * epw

        def body(j, carry):
            pltpu.sync_copy(dst_h.at[pl.ds(base + j * C, C)], didx)
            pltpu.sync_copy(ones_v, hist.at[didx], add=True)
            return carry

        lax.fori_loop(0, nchunks, body, 0)
        plsc.subcore_barrier()
        pltpu.sync_copy(hist.at[pl.ds(sid * rpt, rpt)],
                        out_h.at[cid, pl.ds(sid * rpt, rpt)])

    return k(dst, zeros_h)


def _sc_gcn(table, src, dst, zeros_nd):
    n, d = table.shape
    e = src.shape[0]
    epw = e // NW
    nchunks = epw // C
    rpt = n // NS

    @functools.partial(
        pl.kernel,
        out_type=jax.ShapeDtypeStruct((NC, n, d), jnp.float32),
        mesh=_sc_mesh(),
        scratch_types=[
            pltpu.VMEM((C,), jnp.int32),
            pltpu.VMEM((C,), jnp.int32),
            pltpu.VMEM((C, d), jnp.float32),
            pltpu.VMEM_SHARED((n, d), jnp.float32),
            pltpu.SemaphoreType.DMA,
        ],
    )
    def k(tab_h, src_h, dst_h, z_h, out_h, sidx, didx, rows, acc, sem):
        cid = lax.axis_index("c")
        sid = lax.axis_index("s")
        pltpu.sync_copy(z_h.at[pl.ds(sid * rpt, rpt)], acc.at[pl.ds(sid * rpt, rpt)])
        plsc.subcore_barrier()
        base = (cid * NS + sid) * epw

        def body(j, carry):
            off = base + j * C
            pltpu.sync_copy(src_h.at[pl.ds(off, C)], sidx)
            pltpu.sync_copy(dst_h.at[pl.ds(off, C)], didx)
            pltpu.async_copy(tab_h.at[sidx], rows, sem).wait()
            pltpu.sync_copy(rows, acc.at[didx], add=True)
            return carry

        lax.fori_loop(0, nchunks, body, 0)
        plsc.subcore_barrier()
        r0 = sid * rpt
        for nb in _bounce_chunks(rpt):
            pltpu.sync_copy(acc.at[pl.ds(r0, nb)], rows.at[pl.ds(0, nb)])
            pltpu.sync_copy(rows.at[pl.ds(0, nb)], out_h.at[cid, pl.ds(r0, nb)])
            r0 = r0 + nb

    return k(table, src, dst, zeros_nd)


def _sc_conv(q_t, k_t, v_t, src, dst, zeros_nd, d):
    n, dpad = q_t.shape
    dh = d // H
    scale = 1.0 / math.sqrt(dh)
    e = src.shape[0]
    epw = e // NW
    nchunks = epw // C
    rpt = n // NS

    @functools.partial(
        pl.kernel,
        out_type=jax.ShapeDtypeStruct((NC, n, dpad), jnp.float32),
        mesh=_sc_mesh(),
        compiler_params=pltpu.CompilerParams(needs_layout_passes=False),
        scratch_types=[
            pltpu.VMEM((C,), jnp.int32),
            pltpu.VMEM((C,), jnp.int32),
            pltpu.VMEM((C, dpad), jnp.float32),
            pltpu.VMEM((C, dpad), jnp.float32),
            pltpu.VMEM((C, dpad), jnp.float32),
            pltpu.VMEM((C, dpad), jnp.float32),
            pltpu.VMEM_SHARED((n, dpad), jnp.float32),
            pltpu.SemaphoreType.DMA,
        ],
    )
    def k(q_h, k_h, v_h, src_h, dst_h, z_h, out_h,
          sidx, didx, qr, kr, vr, orows, acc, sem):
        cid = lax.axis_index("c")
        sid = lax.axis_index("s")
        pltpu.sync_copy(z_h.at[pl.ds(sid * rpt, rpt)], acc.at[pl.ds(sid * rpt, rpt)])
        if d < dpad:
            def zrow(e, carry):
                for f in range(d, dpad, 16):
                    orows[e, pl.ds(f, 16)] = jnp.zeros((16,), jnp.float32)
                return carry
            lax.fori_loop(0, C, zrow, 0)
        plsc.subcore_barrier()
        base = (cid * NS + sid) * epw
        iota = lax.iota(jnp.int32, 16)

        def group(g, carry):
            rv = g * 16 + iota
            qT = [plsc.load_gather(qr, [rv, jnp.full((16,), f, jnp.int32)])
                  for f in range(d)]
            kT = [plsc.load_gather(kr, [rv, jnp.full((16,), f, jnp.int32)])
                  for f in range(d)]
            vT = [plsc.load_gather(vr, [rv, jnp.full((16,), f, jnp.int32)])
                  for f in range(d)]
            att = []
            for h in range(H):
                lg = []
                for g2 in range(H):
                    acc_l = qT[h * dh] * kT[g2 * dh]
                    for dd in range(1, dh):
                        acc_l = acc_l + qT[h * dh + dd] * kT[g2 * dh + dd]
                    lg.append(acc_l * scale)
                m = jnp.maximum(jnp.maximum(lg[0], lg[1]),
                                jnp.maximum(lg[2], lg[3]))
                ex = [jnp.exp(l - m) for l in lg]
                s = ex[0] + ex[1] + ex[2] + ex[3]
                att.append([x / s for x in ex])
            for h in range(H):
                for dd in range(dh):
                    o = att[h][0] * vT[dd]
                    for g2 in range(1, H):
                        o = o + att[h][g2] * vT[g2 * dh + dd]
                    plsc.store_scatter(
                        orows, [rv, jnp.full((16,), h * dh + dd, jnp.int32)], o)
            return carry

        def body(j, carry):
            off = base + j * C
            pltpu.sync_copy(src_h.at[pl.ds(off, C)], sidx)
            pltpu.sync_copy(dst_h.at[pl.ds(off, C)], didx)
            pltpu.async_copy(q_h.at[didx], qr, sem).wait()
            pltpu.async_copy(k_h.at[sidx], kr, sem).wait()
            pltpu.async_copy(v_h.at[sidx], vr, sem).wait()
            lax.fori_loop(0, C // 16, group, 0)
            pltpu.sync_copy(orows, acc.at[didx], add=True)
            return carry

        lax.fori_loop(0, nchunks, body, 0)
        plsc.subcore_barrier()
        r0 = sid * rpt
        for nb in _bounce_chunks(rpt):
            pltpu.sync_copy(acc.at[pl.ds(r0, nb)], qr.at[pl.ds(0, nb)])
            pltpu.sync_copy(qr.at[pl.ds(0, nb)], out_h.at[cid, pl.ds(r0, nb)])
            r0 = r0 + nb

    return k(q_t, k_t, v_t, src, dst, zeros_nd)


def _dot(a, b):
    return jnp.dot(a, b, preferred_element_type=jnp.float32)


def _tc_wt_first(x_wt, ne_w, ne_b, lin_w, lin_b, qw, qb, kw, kb, vw, vb):
    def body(x_r, new_r, neb_r, lw_r, lb_r, qw_r, qb_r, kw_r, kb_r, vw_r, vb_r,
             q_o, k_o, v_o):
        x = jax.nn.relu(_dot(x_r[...], new_r[...]) + neb_r[...])
        xl = _dot(x, lw_r[...]) + lb_r[...]
        q_o[...] = _dot(xl, qw_r[...]) + qb_r[...]
        k_o[...] = _dot(xl, kw_r[...]) + kb_r[...]
        v_o[...] = _dot(xl, vw_r[...]) + vb_r[...]

    dout = qw.shape[1]
    outs = [jax.ShapeDtypeStruct((N, dout), jnp.float32)] * 3
    return pl.pallas_call(body, out_shape=outs)(
        x_wt, ne_w, ne_b[None, :], lin_w, lin_b[None, :], qw, qb[None, :],
        kw, kb[None, :], vw, vb[None, :])


def _tc_wt_mid(acc2, lin_w, lin_b, qw, qb, kw, kb, vw, vb):
    def body(a_r, lw_r, lb_r, qw_r, qb_r, kw_r, kb_r, vw_r, vb_r,
             q_o, k_o, v_o):
        x = jax.nn.relu(a_r[0] + a_r[1])
        xl = _dot(x, lw_r[...]) + lb_r[...]
        q_o[...] = _dot(xl, qw_r[...]) + qb_r[...]
        k_o[...] = _dot(xl, kw_r[...]) + kb_r[...]
        v_o[...] = _dot(xl, vw_r[...]) + vb_r[...]

    dout = qw.shape[1]
    outs = [jax.ShapeDtypeStruct((N, dout), jnp.float32)] * 3
    return pl.pallas_call(body, out_shape=outs)(
        acc2, lin_w, lin_b[None, :], qw, qb[None, :], kw, kb[None, :],
        vw, vb[None, :])


def _tc_wt_final(acc2, batch2d, pe, sft_w, sft_b, wih, whh, bih, bhh):
    odim = acc2.shape[2]

    def body(a_r, b_r, pe_r, sw_r, sb_r, wih_r, whh_r, bih_r, bhh_r, g_o):
        x3 = jax.nn.relu(a_r[0] + a_r[1])
        onehot = (jnp.equal(lax.broadcasted_iota(jnp.int32, (NG, N), 0),
                            b_r[...])).astype(jnp.float32)
        cnt = jnp.sum(onehot, axis=1, keepdims=True)
        sm = _dot(onehot, x3) / cnt
        smpe = _dot(onehot, pe_r[...]) / cnt
        sf = jax.nn.relu(_dot(sm, sw_r[...]) + sb_r[...])
        gi = lax.dot_general(sf, wih_r[...], (((1,), (1,)), ((), ())),
                             preferred_element_type=jnp.float32) + bih_r[...]
        i_r = gi[:, 0:odim]
        i_z = gi[:, odim:2 * odim]
        i_n = gi[:, 2 * odim:3 * odim]
        h_r = bhh_r[:, 0:odim]
        h_z = bhh_r[:, odim:2 * odim]
        h_n = bhh_r[:, 2 * odim:3 * odim]
        r = jax.nn.sigmoid(i_r + h_r)
        z = jax.nn.sigmoid(i_z + h_z)
        nn_ = jnp.tanh(i_n + r * h_n)
        sf2 = (1 - z) * nn_
        g_o[...] = jnp.concatenate([sm + smpe, sf2], axis=1)

    out = jax.ShapeDtypeStruct((NG, 2 * odim), jnp.float32)
    return pl.pallas_call(body, out_shape=out)(
        acc2, batch2d, pe, sft_w, sft_b[None, :], wih, whh, bih[None, :],
        bhh[None, :])


def _tc_diff_first(x_diff, hist_t, w0):
    def body(x_r, h_r, w_r, xs_o, dinv_o):
        deg = h_r[0:N, 0:1] + h_r[0:N, 1:2] + 1.0
        dinv = lax.rsqrt(deg)
        dinv_o[...] = dinv
        xs_o[...] = dinv * _dot(x_r[...], w_r[...])

    outs = [jax.ShapeDtypeStruct((N, w0.shape[1]), jnp.float32),
            jax.ShapeDtypeStruct((N, 1), jnp.float32)]
    return pl.pallas_call(body, out_shape=outs)(x_diff, hist_t, w0)


def _tc_diff_mid(acc2, xs_prev, dinv, b_cur, w_next, relu):
    def body(a_r, xp_r, di_r, b_r, w_r, xs_o):
        y = di_r[...] * (a_r[0] + a_r[1] + xp_r[...]) + b_r[...]
        if relu:
            y = jax.nn.relu(y)
        xs_o[...] = di_r[...] * _dot(y, w_r[...])

    out = jax.ShapeDtypeStruct((N, w_next.shape[1]), jnp.float32)
    return pl.pallas_call(body, out_shape=out)(
        acc2, xs_prev, dinv, b_cur[None, :], w_next)


def _tc_final(acc2, xs_prev, dinv, b_cur, batchd2d, g_wt, p):
    def body(a_r, xp_r, di_r, b_r, bd_r, gw_r, m1w_r, m1b_r, m2w_r, m2b_r,
             m3w_r, m3b_r, pw_r, pb_r, out_o):
        y = di_r[...] * (a_r[0] + a_r[1] + xp_r[...]) + b_r[...]
        onehot = (jnp.equal(lax.broadcasted_iota(jnp.int32, (NG, N), 0),
                            bd_r[...])).astype(jnp.float32)
        cnt = jnp.sum(onehot, axis=1, keepdims=True)
        g_diff = _dot(onehot, y) / cnt
        comb = jnp.concatenate([gw_r[...], g_diff], axis=1)
        z = jax.nn.relu(_dot(comb, m1w_r[...]) + m1b_r[...])
        z = jax.nn.relu(_dot(z, m2w_r[...]) + m2b_r[...])
        z = _dot(z, m3w_r[...]) + m3b_r[...]
        out_o[...] = _dot(z, pw_r[...]) + pb_r[...]

    out = jax.ShapeDtypeStruct((NG, 1), jnp.float32)
    return pl.pallas_call(body, out_shape=out)(
        acc2, xs_prev, dinv, b_cur[None, :], batchd2d, g_wt,
        p['mlp1_w'], p['mlp1_b'][None, :], p['mlp2_w'], p['mlp2_b'][None, :],
        p['mlp3_w'], p['mlp3_b'][None, :], p['pred_w'], p['pred_b'][None, :])


def kernel(x_wt, edge_index_wt, batch_wt, x_diff, edge_index_diff, batch_diff, params):
    p = params
    src_wt = edge_index_wt[0].astype(jnp.int32)
    dst_wt = edge_index_wt[1].astype(jnp.int32)
    src_d = edge_index_diff[0].astype(jnp.int32)
    dst_d = edge_index_diff[1].astype(jnp.int32)
    zeros_nd = jnp.zeros((NP, 128), jnp.float32)
    zeros_h = jnp.zeros((NH,), jnp.float32)

    def _pad(x):
        return jnp.pad(x, ((0, NP - N), (0, 128 - x.shape[1])))

    c0 = p['conv0']
    q0, k0, v0 = _tc_wt_first(x_wt, p['ne_w'], p['ne_b'], c0['lin_w'],
                              c0['lin_b'], c0['q_w'], c0['q_b'], c0['k_w'],
                              c0['k_b'], c0['v_w'], c0['v_b'])
    a0 = _sc_conv(_pad(q0), _pad(k0), _pad(v0), src_wt, dst_wt, zeros_nd, 128)
    c1 = p['conv1']
    q1, k1, v1 = _tc_wt_mid(a0[:, :N], c1['lin_w'], c1['lin_b'], c1['q_w'],
                            c1['q_b'], c1['k_w'], c1['k_b'], c1['v_w'],
                            c1['v_b'])
    a1 = _sc_conv(_pad(q1), _pad(k1), _pad(v1), src_wt, dst_wt, zeros_nd, 128)
    c2 = p['conv2']
    q2, k2, v2 = _tc_wt_mid(a1[:, :N], c2['lin_w'], c2['lin_b'], c2['q_w'],
                            c2['q_b'], c2['k_w'], c2['k_b'], c2['v_w'],
                            c2['v_b'])
    a2 = _sc_conv(_pad(q2), _pad(k2), _pad(v2), src_wt, dst_wt, zeros_nd, 64)
    pe = _pos_encoding(N, 64)
    g_wt = _tc_wt_final(a2[:, :N, :64], batch_wt.astype(jnp.int32)[None, :], pe,
                        p['sft_w'], p['sft_b'], p['gru_wih'], p['gru_whh'],
                        p['gru_bih'], p['gru_bhh'])

    hist = _sc_deg(dst_d, zeros_h)
    xs0, dinv = _tc_diff_first(x_diff, hist.T, p['gcn0_w'])
    ad0 = _sc_gcn(_pad(xs0), src_d, dst_d, zeros_nd)
    xs1 = _tc_diff_mid(ad0[:, :N], xs0, dinv, p['gcn0_b'], p['gcn1_w'], True)
    ad1 = _sc_gcn(_pad(xs1), src_d, dst_d, zeros_nd)
    xs2 = _tc_diff_mid(ad1[:, :N], xs1, dinv, p['gcn1_b'], p['gcn2_w'], True)
    ad2 = _sc_gcn(_pad(xs2), src_d, dst_d, zeros_nd)
    return _tc_final(ad2[:, :N], xs2, dinv, p['gcn2_b'],
                     batch_diff.astype(jnp.int32)[None, :], g_wt, p)
